# baseline (speedup 1.0000x reference)
import jax
import jax.numpy as jnp
from jax import lax
from jax.experimental import pallas as pl
from jax.experimental.pallas import tpu as pltpu

_N = 16777216
_BLK = 524288
_GRID = _N // _BLK


def _fill_body(idx_ref, grad_ref, out_ref):
    pid = pl.program_id(0)
    target = idx_ref[0]
    kstar = target // _BLK
    off = target % _BLK

    @pl.when(kstar != pid)
    def _():
        out_ref[...] = jnp.zeros_like(out_ref)

    @pl.when(kstar == pid)
    def _():
        pos = lax.broadcasted_iota(jnp.int32, (_BLK,), 0)
        out_ref[...] = jnp.where(pos == off, grad_ref[0], 0.0)


def kernel(grad_output, input_sizes, dim, index, out):
    n = out.shape[0]
    idx = ((jnp.asarray(index, jnp.int32) + jnp.asarray(dim, jnp.int32))
           % jnp.asarray(input_sizes, jnp.int32)).reshape((1,))
    gval = jnp.asarray(grad_output, jnp.float32).reshape((1,))
    res = pl.pallas_call(
        _fill_body,
        grid=(_GRID,),
        in_specs=[pl.BlockSpec(memory_space=pltpu.SMEM),
                  pl.BlockSpec(memory_space=pltpu.SMEM)],
        out_specs=pl.BlockSpec((_BLK,), lambda i: (i,)),
        out_shape=jax.ShapeDtypeStruct((n,), jnp.float32),
    )(idx, gval)
    return res
